# Initial kernel scaffold; baseline (speedup 1.0000x reference)
import jax
import jax.numpy as jnp
from jax.experimental import pallas as pl


def kernel(xs, embedding, W):
    raise NotImplementedError("write your pallas kernel here")


# speedup vs baseline: 2.1704x; 2.1704x over previous
import functools

import jax
import jax.numpy as jnp
from jax import lax
from jax.experimental import pallas as pl
from jax.experimental.pallas import tpu as pltpu
from jax.experimental.pallas import tpu_sc as plsc

VOCAB = 27
EMB = 5
OUT = 27
PAD = 32
BATCH = 16384

NC = 2
NS = 16
NW = NC * NS
B_PER_W = BATCH // NW
CHUNK = 128
NCHUNK = B_PER_W // CHUNK
LANES = 16


def _table_body(emb_ref, w_ref, out_ref):
    emb = emb_ref[...]
    w = w_ref[...]
    t1 = jnp.dot(emb, w[0:EMB, :], preferred_element_type=jnp.float32,
                 precision=lax.Precision.HIGHEST)
    t2 = jnp.dot(emb, w[EMB:, :], preferred_element_type=jnp.float32,
                 precision=lax.Precision.HIGHEST)
    for a in range(VOCAB):
        out_ref[pl.ds(a * VOCAB, VOCAB), :] = t1[a:a + 1, :] + t2


_build_table = pl.pallas_call(
    _table_body,
    out_shape=jax.ShapeDtypeStruct((VOCAB * VOCAB, PAD), jnp.float32),
)


def _sc_body(table_hbm, xs_hbm, out_hbm, xs_v, idx_v, rows_v, sem):
    wid = lax.axis_index("s") * NC + lax.axis_index("c")
    base = wid * B_PER_W
    pltpu.sync_copy(xs_hbm.at[pl.ds(base * 2, B_PER_W * 2)], xs_v)
    i16 = lax.iota(jnp.int32, LANES)
    copies = []
    for c in range(NCHUNK):
        for j in range(CHUNK // LANES):
            off = (c * CHUNK + j * LANES) * 2
            x0 = plsc.load_gather(xs_v, [i16 * 2 + off])
            x1 = plsc.load_gather(xs_v, [i16 * 2 + (off + 1)])
            idx_v[c, pl.ds(j * LANES, LANES)] = x0 * VOCAB + x1
        copies.append(
            pltpu.async_copy(
                table_hbm.at[idx_v.at[c]],
                rows_v.at[pl.ds(c * CHUNK, CHUNK)],
                sem,
            )
        )
    for cp in copies:
        cp.wait()
    pltpu.sync_copy(rows_v, out_hbm.at[pl.ds(base, B_PER_W)])


@functools.lru_cache(maxsize=None)
def _make_gather():
    return pl.kernel(
        _sc_body,
        out_type=jax.ShapeDtypeStruct((BATCH, PAD), jnp.float32),
        mesh=plsc.VectorSubcoreMesh(core_axis_name="c", subcore_axis_name="s"),
        compiler_params=pltpu.CompilerParams(
            needs_layout_passes=False, use_tc_tiling_on_sc=False
        ),
        scratch_types=[
            pltpu.VMEM((2 * B_PER_W,), jnp.int32),
            pltpu.VMEM((NCHUNK, CHUNK), jnp.int32),
            pltpu.VMEM((B_PER_W, PAD), jnp.float32),
            pltpu.SemaphoreType.DMA,
        ],
    )


def kernel(xs, embedding, W):
    w_pad = jnp.zeros((2 * EMB, PAD), jnp.float32).at[:, :OUT].set(W)
    table = _build_table(embedding, w_pad)
    out = _make_gather()(table, xs.reshape(-1))
    return out[:, :OUT]
